# Initial kernel scaffold; baseline (speedup 1.0000x reference)
import jax
import jax.numpy as jnp
from jax.experimental import pallas as pl


def kernel(x, prototypes, prototype_outputs, relevance):
    raise NotImplementedError("write your pallas kernel here")


# speedup vs baseline: 1.8257x; 1.8257x over previous
import jax
import jax.numpy as jnp
from jax.experimental import pallas as pl
from jax.experimental.pallas import tpu as pltpu

_TILE_B = 512


def _nearest_kernel(x_ref, pt_ref, w_ref, pout_ref, out_ref):
    tb = x_ref.shape[0]
    d_feat = x_ref.shape[1]
    n_proto = pt_ref.shape[1]
    od = pout_ref.shape[0]

    acc = None
    for d in range(d_feat):
        t = x_ref[:, d : d + 1] - pt_ref[d : d + 1, :]
        term = (t * t) * w_ref[0, d]
        acc = term if acc is None else acc + term

    m = jnp.min(acc, axis=1, keepdims=True)
    iota = jax.lax.broadcasted_iota(jnp.int32, (tb, n_proto), 1)
    cand = jnp.where(acc == m, iota, n_proto)
    win = jnp.min(cand, axis=1, keepdims=True)
    onehot = iota == win
    for j in range(od):
        sel = jnp.where(onehot, pout_ref[j : j + 1, :], 0.0)
        out_ref[:, j : j + 1] = jnp.sum(sel, axis=1, keepdims=True)


def kernel(x, prototypes, prototype_outputs, relevance):
    b, d_feat = x.shape
    n_proto = prototypes.shape[0]
    od = prototype_outputs.shape[1]

    w = (relevance * relevance).reshape(1, d_feat)
    pt = prototypes.T
    pout = prototype_outputs.T

    tile_b = min(_TILE_B, b)
    grid = (b // tile_b,)

    out = pl.pallas_call(
        _nearest_kernel,
        grid=grid,
        in_specs=[
            pl.BlockSpec((tile_b, d_feat), lambda i: (i, 0)),
            pl.BlockSpec((d_feat, n_proto), lambda i: (0, 0)),
            pl.BlockSpec((1, d_feat), lambda i: (0, 0)),
            pl.BlockSpec((od, n_proto), lambda i: (0, 0)),
        ],
        out_specs=pl.BlockSpec((tile_b, od), lambda i: (i, 0)),
        out_shape=jax.ShapeDtypeStruct((b, od), jnp.float32),
        compiler_params=pltpu.CompilerParams(
            dimension_semantics=("parallel",),
        ),
    )(x, pt, w, pout)
    return out


# speedup vs baseline: 2.0910x; 1.1453x over previous
import dataclasses
import functools

import jax
import jax.numpy as jnp
from jax import lax
from jax.experimental import pallas as pl
from jax.experimental.pallas import tpu as pltpu
from jax.experimental.pallas import tpu_sc as plsc

_TILE_B = 512


def _top2_kernel(x_ref, pt_ref, wcol_ref, win1_ref, win2_ref):
    tb = x_ref.shape[0]
    n_proto = pt_ref.shape[1]

    wpt = pt_ref[...] * wcol_ref[...]
    pnorm = jnp.sum(pt_ref[...] * wpt, axis=0, keepdims=True)
    score = jax.lax.dot_general(
        x_ref[...], wpt, (((1,), (0,)), ((), ())),
        precision=jax.lax.Precision.HIGHEST,
        preferred_element_type=jnp.float32,
    )
    g = pnorm - 2.0 * score

    iota = jax.lax.broadcasted_iota(jnp.int32, (tb, n_proto), 1)
    m1 = jnp.min(g, axis=1, keepdims=True)
    cand1 = jnp.where(g == m1, iota, n_proto)
    w1 = jnp.min(cand1, axis=1, keepdims=True)
    gm = jnp.where(iota == w1, jnp.inf, g)
    m2 = jnp.min(gm, axis=1, keepdims=True)
    cand2 = jnp.where(gm == m2, iota, n_proto)
    w2 = jnp.min(cand2, axis=1, keepdims=True)
    win1_ref[...] = w1
    win2_ref[...] = w2


def _sc_refine(n_chunk, d_feat, protos_hbm, pout_hbm, xr_hbm, idx1_hbm,
               idx2_hbm, wsp_hbm, out_hbm, rows1_v, rows2_v, pout_v, xr_v,
               idx1_v, idx2_v, wsp_v, out_v, sem1, sem2):
    nc = jax.lax.axis_index("c")
    ns = jax.lax.axis_index("s")
    wid = ns * 2 + nc
    base = wid * n_chunk

    pltpu.sync_copy(idx1_hbm.at[pl.ds(base, n_chunk)], idx1_v)
    pltpu.sync_copy(idx2_hbm.at[pl.ds(base, n_chunk)], idx2_v)
    cp1 = pltpu.async_copy(protos_hbm.at[idx1_v], rows1_v, sem1)
    cp2 = pltpu.async_copy(protos_hbm.at[idx2_v], rows2_v, sem2)
    pltpu.sync_copy(pout_hbm, pout_v)
    pltpu.sync_copy(wsp_hbm, wsp_v)
    pltpu.sync_copy(xr_hbm.at[wid], xr_v)
    cp1.wait()
    cp2.wait()

    @pl.loop(0, n_chunk // 16)
    def _group(i):
        off = pl.multiple_of(i * 16, 16)
        sl = pl.ds(off, 16)
        i1 = idx1_v[sl]
        i2 = idx2_v[sl]
        qloc = lax.iota(jnp.int32, 16) + off
        e1 = jnp.zeros((16,), jnp.float32)
        e2 = jnp.zeros((16,), jnp.float32)
        for d in range(d_feat):
            xd = xr_v[d, sl]
            wd = wsp_v[d]
            dcol = jnp.full((16,), d, jnp.int32)
            p1d = plsc.load_gather(rows1_v, [qloc, dcol])
            p2d = plsc.load_gather(rows2_v, [qloc, dcol])
            t1 = xd - p1d
            t2 = xd - p2d
            e1 = e1 + (t1 * t1) * wd
            e2 = e2 + (t2 * t2) * wd
        take2 = (e2 < e1) | ((e2 == e1) & (i2 < i1))
        ch = jnp.where(take2, i2, i1)
        out_v[sl] = plsc.load_gather(pout_v, [ch])

    pltpu.sync_copy(out_v, out_hbm.at[pl.ds(base, n_chunk)])


def kernel(x, prototypes, prototype_outputs, relevance):
    b, d_feat = x.shape
    n_proto = prototypes.shape[0]
    od = prototype_outputs.shape[1]

    w = relevance * relevance
    pt = prototypes.T
    wcol = w.reshape(d_feat, 1)

    tile_b = min(_TILE_B, b)
    grid = (b // tile_b,)
    win1, win2 = pl.pallas_call(
        _top2_kernel,
        grid=grid,
        in_specs=[
            pl.BlockSpec((tile_b, d_feat), lambda i: (i, 0)),
            pl.BlockSpec((d_feat, n_proto), lambda i: (0, 0)),
            pl.BlockSpec((d_feat, 1), lambda i: (0, 0)),
        ],
        out_specs=[
            pl.BlockSpec((tile_b, 1), lambda i: (i, 0)),
            pl.BlockSpec((tile_b, 1), lambda i: (i, 0)),
        ],
        out_shape=[
            jax.ShapeDtypeStruct((b, 1), jnp.int32),
            jax.ShapeDtypeStruct((b, 1), jnp.int32),
        ],
        compiler_params=pltpu.CompilerParams(
            dimension_semantics=("parallel",),
        ),
    )(x, pt, wcol)

    n_workers = 32
    n_chunk = b // n_workers
    xr = x.reshape(n_workers, n_chunk, d_feat).transpose(0, 2, 1)
    wsp = jnp.broadcast_to(w.reshape(d_feat, 1), (d_feat, 16))
    pout_flat = prototype_outputs[:, 0]

    sc_params = pltpu.CompilerParams()
    fields = pltpu.CompilerParams.__dataclass_fields__
    if "needs_layout_passes" in fields:
        sc_params = dataclasses.replace(sc_params, needs_layout_passes=False)
    if "use_tc_tiling_on_sc" in fields:
        sc_params = dataclasses.replace(sc_params, use_tc_tiling_on_sc=False)
    mesh = plsc.VectorSubcoreMesh(core_axis_name="c", subcore_axis_name="s")
    refine = pl.kernel(
        functools.partial(_sc_refine, n_chunk, d_feat),
        out_type=jax.ShapeDtypeStruct((b,), jnp.float32),
        mesh=mesh,
        scratch_types=[
            pltpu.VMEM((n_chunk, d_feat), jnp.float32),
            pltpu.VMEM((n_chunk, d_feat), jnp.float32),
            pltpu.VMEM((n_proto,), jnp.float32),
            pltpu.VMEM((d_feat, n_chunk), jnp.float32),
            pltpu.VMEM((n_chunk,), jnp.int32),
            pltpu.VMEM((n_chunk,), jnp.int32),
            pltpu.VMEM((d_feat, 16), jnp.float32),
            pltpu.VMEM((n_chunk,), jnp.float32),
            pltpu.SemaphoreType.DMA,
            pltpu.SemaphoreType.DMA,
        ],
        compiler_params=sc_params,
    )
    out_flat = refine(prototypes, pout_flat, xr, win1.reshape(b),
                      win2.reshape(b), wsp)
    return out_flat.reshape(b, od)


# speedup vs baseline: 2.1332x; 1.0202x over previous
import dataclasses
import functools

import jax
import jax.numpy as jnp
from jax import lax
from jax.experimental import pallas as pl
from jax.experimental.pallas import tpu as pltpu
from jax.experimental.pallas import tpu_sc as plsc

_TILE_B = 512
_N_WORKERS = 32


def _top2_kernel(x_ref, pt_ref, wcol_ref, win_ref, xr_ref):
    tb = x_ref.shape[0]
    n_proto = pt_ref.shape[1]
    n_chunk = xr_ref.shape[2]

    wpt = pt_ref[...] * wcol_ref[...]
    pnorm = jnp.sum(pt_ref[...] * wpt, axis=0, keepdims=True)
    score = jax.lax.dot_general(
        x_ref[...], wpt, (((1,), (0,)), ((), ())),
        precision=jax.lax.Precision.HIGHEST,
        preferred_element_type=jnp.float32,
    )
    g = pnorm - 2.0 * score

    iota = jax.lax.broadcasted_iota(jnp.int32, (tb, n_proto), 1)
    m1 = jnp.min(g, axis=1, keepdims=True)
    cand1 = jnp.where(g == m1, iota, n_proto)
    w1 = jnp.min(cand1, axis=1, keepdims=True)
    gm = jnp.where(iota == w1, jnp.inf, g)
    m2 = jnp.min(gm, axis=1, keepdims=True)
    cand2 = jnp.where(gm == m2, iota, n_proto)
    w2 = jnp.min(cand2, axis=1, keepdims=True)

    win_ref[0, 0:1, :] = w1.T
    win_ref[0, 1:2, :] = w2.T

    xt = x_ref[...].T
    d_feat = x_ref.shape[1]
    for c in range(tb // n_chunk):
        xr_ref[c] = xt[:, c * n_chunk : (c + 1) * n_chunk]


def _sc_refine(n_chunk, d_feat, chunks_per_tile, protos_hbm, pout_hbm,
               xr_hbm, win_hbm, rel_hbm, out_hbm, rows1_v, rows2_v, pout_v,
               xr_v, idx1_v, idx2_v, w_v, out_v, sem1, sem2):
    nc = jax.lax.axis_index("c")
    ns = jax.lax.axis_index("s")
    wid = ns * 2 + nc
    tile = wid // chunks_per_tile
    cof = (wid % chunks_per_tile) * n_chunk

    pltpu.sync_copy(win_hbm.at[tile, 0, pl.ds(cof, n_chunk)], idx1_v)
    pltpu.sync_copy(win_hbm.at[tile, 1, pl.ds(cof, n_chunk)], idx2_v)
    cp1 = pltpu.async_copy(protos_hbm.at[idx1_v], rows1_v, sem1)
    cp2 = pltpu.async_copy(protos_hbm.at[idx2_v], rows2_v, sem2)
    pltpu.sync_copy(pout_hbm, pout_v)
    pltpu.sync_copy(rel_hbm, w_v)
    pltpu.sync_copy(xr_hbm.at[wid], xr_v)
    rel = w_v[...]
    w_v[...] = rel * rel
    cp1.wait()
    cp2.wait()

    zeros16 = jnp.zeros((16,), jnp.int32)

    @pl.loop(0, n_chunk // 16)
    def _group(i):
        off = pl.multiple_of(i * 16, 16)
        sl = pl.ds(off, 16)
        i1 = idx1_v[sl]
        i2 = idx2_v[sl]
        qloc = lax.iota(jnp.int32, 16) + off
        e1 = jnp.zeros((16,), jnp.float32)
        e2 = jnp.zeros((16,), jnp.float32)
        for d in range(d_feat):
            xd = xr_v[d, sl]
            dcol = jnp.full((16,), d, jnp.int32)
            wd = plsc.load_gather(w_v, [dcol])
            p1d = plsc.load_gather(rows1_v, [qloc, dcol])
            p2d = plsc.load_gather(rows2_v, [qloc, dcol])
            t1 = xd - p1d
            t2 = xd - p2d
            e1 = e1 + (t1 * t1) * wd
            e2 = e2 + (t2 * t2) * wd
        take2 = (e2 < e1) | ((e2 == e1) & (i2 < i1))
        ch = jnp.where(take2, i2, i1)
        out_v[sl] = plsc.load_gather(pout_v, [ch, zeros16])

    base = wid * n_chunk
    pltpu.sync_copy(out_v, out_hbm.at[pl.ds(base, n_chunk)])


def kernel(x, prototypes, prototype_outputs, relevance):
    b, d_feat = x.shape
    n_proto = prototypes.shape[0]
    od = prototype_outputs.shape[1]

    w = relevance * relevance
    pt = prototypes.T
    wcol = w.reshape(d_feat, 1)

    tile_b = min(_TILE_B, b)
    n_tiles = b // tile_b
    n_chunk = b // _N_WORKERS
    chunks_per_tile = tile_b // n_chunk

    win, xr = pl.pallas_call(
        _top2_kernel,
        grid=(n_tiles,),
        in_specs=[
            pl.BlockSpec((tile_b, d_feat), lambda i: (i, 0)),
            pl.BlockSpec((d_feat, n_proto), lambda i: (0, 0)),
            pl.BlockSpec((d_feat, 1), lambda i: (0, 0)),
        ],
        out_specs=[
            pl.BlockSpec((1, 2, tile_b), lambda i: (i, 0, 0)),
            pl.BlockSpec((chunks_per_tile, d_feat, n_chunk),
                         lambda i: (i, 0, 0)),
        ],
        out_shape=[
            jax.ShapeDtypeStruct((n_tiles, 2, tile_b), jnp.int32),
            jax.ShapeDtypeStruct((_N_WORKERS, d_feat, n_chunk), jnp.float32),
        ],
        compiler_params=pltpu.CompilerParams(
            dimension_semantics=("parallel",),
        ),
    )(x, pt, wcol)

    sc_params = pltpu.CompilerParams()
    fields = pltpu.CompilerParams.__dataclass_fields__
    if "needs_layout_passes" in fields:
        sc_params = dataclasses.replace(sc_params, needs_layout_passes=False)
    if "use_tc_tiling_on_sc" in fields:
        sc_params = dataclasses.replace(sc_params, use_tc_tiling_on_sc=False)
    mesh = plsc.VectorSubcoreMesh(core_axis_name="c", subcore_axis_name="s")
    refine = pl.kernel(
        functools.partial(_sc_refine, n_chunk, d_feat, chunks_per_tile),
        out_type=jax.ShapeDtypeStruct((b,), jnp.float32),
        mesh=mesh,
        scratch_types=[
            pltpu.VMEM((n_chunk, d_feat), jnp.float32),
            pltpu.VMEM((n_chunk, d_feat), jnp.float32),
            pltpu.VMEM((n_proto, od), jnp.float32),
            pltpu.VMEM((d_feat, n_chunk), jnp.float32),
            pltpu.VMEM((n_chunk,), jnp.int32),
            pltpu.VMEM((n_chunk,), jnp.int32),
            pltpu.VMEM((d_feat,), jnp.float32),
            pltpu.VMEM((n_chunk,), jnp.float32),
            pltpu.SemaphoreType.DMA,
            pltpu.SemaphoreType.DMA,
        ],
        compiler_params=sc_params,
    )
    out_flat = refine(prototypes, prototype_outputs, xr, win, relevance)
    return out_flat.reshape(b, od)


# speedup vs baseline: 2.2154x; 1.0385x over previous
import dataclasses
import functools

import jax
import jax.numpy as jnp
from jax import lax
from jax.experimental import pallas as pl
from jax.experimental.pallas import tpu as pltpu
from jax.experimental.pallas import tpu_sc as plsc

_TILE_B = 512
_N_WORKERS = 32


def _top2_kernel(x_ref, pt_ref, wcol_ref, win_ref, xr_ref):
    tb = x_ref.shape[0]
    n_proto = pt_ref.shape[1]
    n_chunk = xr_ref.shape[2]

    wpt = pt_ref[...] * wcol_ref[...]
    pnorm = jnp.sum(pt_ref[...] * wpt, axis=0, keepdims=True)
    score = jax.lax.dot_general(
        x_ref[...], wpt, (((1,), (0,)), ((), ())),
        precision=jax.lax.Precision.HIGHEST,
        preferred_element_type=jnp.float32,
    )
    g = pnorm - 2.0 * score

    iota = jax.lax.broadcasted_iota(jnp.int32, (tb, n_proto), 1)
    m1 = jnp.min(g, axis=1, keepdims=True)
    cand1 = jnp.where(g == m1, iota, n_proto)
    w1 = jnp.min(cand1, axis=1, keepdims=True)
    gm = jnp.where(iota == w1, jnp.inf, g)
    m2 = jnp.min(gm, axis=1, keepdims=True)
    cand2 = jnp.where(gm == m2, iota, n_proto)
    w2 = jnp.min(cand2, axis=1, keepdims=True)

    win_ref[0, 0:1, :] = w1.T
    win_ref[0, 1:2, :] = w2.T

    xt = x_ref[...].T
    d_feat = x_ref.shape[1]
    for c in range(tb // n_chunk):
        xr_ref[c] = xt[:, c * n_chunk : (c + 1) * n_chunk]


def _sc_refine(n_chunk, d_feat, chunks_per_tile, protos_hbm, pout_hbm,
               xr_hbm, win_hbm, rel_hbm, out_hbm, rows1_v, rows2_v, pout_v,
               xr_v, idx1_v, idx2_v, w_v, out_v, sem1, sem2, sem3, sem4,
               sem5):
    nc = jax.lax.axis_index("c")
    ns = jax.lax.axis_index("s")
    wid = ns * 2 + nc
    tile = wid // chunks_per_tile
    cof = (wid % chunks_per_tile) * n_chunk

    cpa = pltpu.async_copy(win_hbm.at[tile, 0, pl.ds(cof, n_chunk)], idx1_v,
                           sem1)
    cpb = pltpu.async_copy(win_hbm.at[tile, 1, pl.ds(cof, n_chunk)], idx2_v,
                           sem2)
    cpc = pltpu.async_copy(pout_hbm, pout_v, sem3)
    cpd = pltpu.async_copy(rel_hbm, w_v, sem4)
    cpe = pltpu.async_copy(xr_hbm.at[wid], xr_v, sem5)
    cpa.wait()
    cpb.wait()
    cp1 = pltpu.async_copy(protos_hbm.at[idx1_v], rows1_v, sem1)
    cp2 = pltpu.async_copy(protos_hbm.at[idx2_v], rows2_v, sem2)
    cpd.wait()
    rel = w_v[...]
    w_v[...] = rel * rel
    wds = [plsc.load_gather(w_v, [jnp.full((16,), d, jnp.int32)])
           for d in range(d_feat)]
    cpc.wait()
    cpe.wait()
    cp1.wait()
    cp2.wait()

    zeros16 = jnp.zeros((16,), jnp.int32)

    @pl.loop(0, n_chunk // 16)
    def _group(i):
        off = pl.multiple_of(i * 16, 16)
        sl = pl.ds(off, 16)
        i1 = idx1_v[sl]
        i2 = idx2_v[sl]
        qloc = lax.iota(jnp.int32, 16) + off
        e1 = jnp.zeros((16,), jnp.float32)
        e2 = jnp.zeros((16,), jnp.float32)
        for d in range(d_feat):
            xd = xr_v[d, sl]
            dcol = jnp.full((16,), d, jnp.int32)
            p1d = plsc.load_gather(rows1_v, [qloc, dcol])
            p2d = plsc.load_gather(rows2_v, [qloc, dcol])
            t1 = xd - p1d
            t2 = xd - p2d
            e1 = e1 + (t1 * t1) * wds[d]
            e2 = e2 + (t2 * t2) * wds[d]
        take2 = (e2 < e1) | ((e2 == e1) & (i2 < i1))
        ch = jnp.where(take2, i2, i1)
        out_v[sl] = plsc.load_gather(pout_v, [ch, zeros16])

    base = wid * n_chunk
    pltpu.sync_copy(out_v, out_hbm.at[pl.ds(base, n_chunk)])


def kernel(x, prototypes, prototype_outputs, relevance):
    b, d_feat = x.shape
    n_proto = prototypes.shape[0]
    od = prototype_outputs.shape[1]

    w = relevance * relevance
    pt = prototypes.T
    wcol = w.reshape(d_feat, 1)

    tile_b = min(_TILE_B, b)
    n_tiles = b // tile_b
    n_chunk = b // _N_WORKERS
    chunks_per_tile = tile_b // n_chunk

    win, xr = pl.pallas_call(
        _top2_kernel,
        grid=(n_tiles,),
        in_specs=[
            pl.BlockSpec((tile_b, d_feat), lambda i: (i, 0)),
            pl.BlockSpec((d_feat, n_proto), lambda i: (0, 0)),
            pl.BlockSpec((d_feat, 1), lambda i: (0, 0)),
        ],
        out_specs=[
            pl.BlockSpec((1, 2, tile_b), lambda i: (i, 0, 0)),
            pl.BlockSpec((chunks_per_tile, d_feat, n_chunk),
                         lambda i: (i, 0, 0)),
        ],
        out_shape=[
            jax.ShapeDtypeStruct((n_tiles, 2, tile_b), jnp.int32),
            jax.ShapeDtypeStruct((_N_WORKERS, d_feat, n_chunk), jnp.float32),
        ],
        compiler_params=pltpu.CompilerParams(
            dimension_semantics=("parallel",),
        ),
    )(x, pt, wcol)

    sc_params = pltpu.CompilerParams()
    fields = pltpu.CompilerParams.__dataclass_fields__
    if "needs_layout_passes" in fields:
        sc_params = dataclasses.replace(sc_params, needs_layout_passes=False)
    if "use_tc_tiling_on_sc" in fields:
        sc_params = dataclasses.replace(sc_params, use_tc_tiling_on_sc=False)
    mesh = plsc.VectorSubcoreMesh(core_axis_name="c", subcore_axis_name="s")
    refine = pl.kernel(
        functools.partial(_sc_refine, n_chunk, d_feat, chunks_per_tile),
        out_type=jax.ShapeDtypeStruct((b,), jnp.float32),
        mesh=mesh,
        scratch_types=[
            pltpu.VMEM((n_chunk, d_feat), jnp.float32),
            pltpu.VMEM((n_chunk, d_feat), jnp.float32),
            pltpu.VMEM((n_proto, od), jnp.float32),
            pltpu.VMEM((d_feat, n_chunk), jnp.float32),
            pltpu.VMEM((n_chunk,), jnp.int32),
            pltpu.VMEM((n_chunk,), jnp.int32),
            pltpu.VMEM((d_feat,), jnp.float32),
            pltpu.VMEM((n_chunk,), jnp.float32),
            pltpu.SemaphoreType.DMA,
            pltpu.SemaphoreType.DMA,
            pltpu.SemaphoreType.DMA,
            pltpu.SemaphoreType.DMA,
            pltpu.SemaphoreType.DMA,
        ],
        compiler_params=sc_params,
    )
    out_flat = refine(prototypes, prototype_outputs, xr, win, relevance)
    return out_flat.reshape(b, od)


# speedup vs baseline: 2.4703x; 1.1151x over previous
import dataclasses
import functools

import jax
import jax.numpy as jnp
from jax import lax
from jax.experimental import pallas as pl
from jax.experimental.pallas import tpu as pltpu
from jax.experimental.pallas import tpu_sc as plsc

_TILE_B = 512
_N_WORKERS = 32


def _top2_kernel(x_ref, pt_ref, wcol_ref, win_ref, xr_ref):
    tb = x_ref.shape[0]
    n_proto = pt_ref.shape[1]
    n_chunk = xr_ref.shape[2]

    wpt = pt_ref[...] * wcol_ref[...]
    pnorm = jnp.sum(pt_ref[...] * wpt, axis=0, keepdims=True)
    xf = x_ref[...]
    xh = xf.astype(jnp.bfloat16)
    xl = (xf - xh.astype(jnp.float32)).astype(jnp.bfloat16)
    wh = wpt.astype(jnp.bfloat16)
    wl = (wpt - wh.astype(jnp.float32)).astype(jnp.bfloat16)
    dn = (((1,), (0,)), ((), ()))
    mm = functools.partial(jax.lax.dot_general, dimension_numbers=dn,
                           preferred_element_type=jnp.float32)
    score = mm(xh, wh) + (mm(xh, wl) + mm(xl, wh)) + mm(xl, wl)
    g = pnorm - 2.0 * score

    iota = jax.lax.broadcasted_iota(jnp.int32, (tb, n_proto), 1)
    w1 = jnp.argmin(g, axis=1).astype(jnp.int32)[:, None]
    gm = jnp.where(iota == w1, jnp.inf, g)
    w2 = jnp.argmin(gm, axis=1).astype(jnp.int32)[:, None]

    win_ref[0, 0:1, :] = w1.T
    win_ref[0, 1:2, :] = w2.T

    xt = x_ref[...].T
    d_feat = x_ref.shape[1]
    for c in range(tb // n_chunk):
        xr_ref[c] = xt[:, c * n_chunk : (c + 1) * n_chunk]


def _sc_refine(n_chunk, d_feat, chunks_per_tile, protos_hbm, pout_hbm,
               xr_hbm, win_hbm, rel_hbm, out_hbm, rows1_v, rows2_v, pout_v,
               xr_v, idx1_v, idx2_v, w_v, out_v, sem1, sem2, sem3, sem4,
               sem5):
    nc = jax.lax.axis_index("c")
    ns = jax.lax.axis_index("s")
    wid = ns * 2 + nc
    tile = wid // chunks_per_tile
    cof = (wid % chunks_per_tile) * n_chunk

    cpa = pltpu.async_copy(win_hbm.at[tile, 0, pl.ds(cof, n_chunk)], idx1_v,
                           sem1)
    cpb = pltpu.async_copy(win_hbm.at[tile, 1, pl.ds(cof, n_chunk)], idx2_v,
                           sem2)
    cpc = pltpu.async_copy(pout_hbm, pout_v, sem3)
    cpd = pltpu.async_copy(rel_hbm, w_v, sem4)
    cpe = pltpu.async_copy(xr_hbm.at[wid], xr_v, sem5)
    cpa.wait()
    cpb.wait()
    cp1 = pltpu.async_copy(protos_hbm.at[idx1_v], rows1_v, sem1)
    cp2 = pltpu.async_copy(protos_hbm.at[idx2_v], rows2_v, sem2)
    cpd.wait()
    rel = w_v[...]
    w_v[...] = rel * rel
    wds = [plsc.load_gather(w_v, [jnp.full((16,), d, jnp.int32)])
           for d in range(d_feat)]
    cpc.wait()
    cpe.wait()
    cp1.wait()
    cp2.wait()

    zeros16 = jnp.zeros((16,), jnp.int32)

    @pl.loop(0, n_chunk // 16)
    def _group(i):
        off = pl.multiple_of(i * 16, 16)
        sl = pl.ds(off, 16)
        i1 = idx1_v[sl]
        i2 = idx2_v[sl]
        qloc = lax.iota(jnp.int32, 16) + off
        e1 = jnp.zeros((16,), jnp.float32)
        e2 = jnp.zeros((16,), jnp.float32)
        for d in range(d_feat):
            xd = xr_v[d, sl]
            dcol = jnp.full((16,), d, jnp.int32)
            p1d = plsc.load_gather(rows1_v, [qloc, dcol])
            p2d = plsc.load_gather(rows2_v, [qloc, dcol])
            t1 = xd - p1d
            t2 = xd - p2d
            e1 = e1 + (t1 * t1) * wds[d]
            e2 = e2 + (t2 * t2) * wds[d]
        take2 = (e2 < e1) | ((e2 == e1) & (i2 < i1))
        ch = jnp.where(take2, i2, i1)
        out_v[sl] = plsc.load_gather(pout_v, [ch, zeros16])

    base = wid * n_chunk
    pltpu.sync_copy(out_v, out_hbm.at[pl.ds(base, n_chunk)])


def kernel(x, prototypes, prototype_outputs, relevance):
    b, d_feat = x.shape
    n_proto = prototypes.shape[0]
    od = prototype_outputs.shape[1]

    w = relevance * relevance
    pt = prototypes.T
    wcol = w.reshape(d_feat, 1)

    tile_b = min(_TILE_B, b)
    n_tiles = b // tile_b
    n_chunk = b // _N_WORKERS
    chunks_per_tile = tile_b // n_chunk

    win, xr = pl.pallas_call(
        _top2_kernel,
        grid=(n_tiles,),
        in_specs=[
            pl.BlockSpec((tile_b, d_feat), lambda i: (i, 0)),
            pl.BlockSpec((d_feat, n_proto), lambda i: (0, 0)),
            pl.BlockSpec((d_feat, 1), lambda i: (0, 0)),
        ],
        out_specs=[
            pl.BlockSpec((1, 2, tile_b), lambda i: (i, 0, 0)),
            pl.BlockSpec((chunks_per_tile, d_feat, n_chunk),
                         lambda i: (i, 0, 0)),
        ],
        out_shape=[
            jax.ShapeDtypeStruct((n_tiles, 2, tile_b), jnp.int32),
            jax.ShapeDtypeStruct((_N_WORKERS, d_feat, n_chunk), jnp.float32),
        ],
        compiler_params=pltpu.CompilerParams(
            dimension_semantics=("parallel",),
        ),
    )(x, pt, wcol)

    sc_params = pltpu.CompilerParams()
    fields = pltpu.CompilerParams.__dataclass_fields__
    if "needs_layout_passes" in fields:
        sc_params = dataclasses.replace(sc_params, needs_layout_passes=False)
    if "use_tc_tiling_on_sc" in fields:
        sc_params = dataclasses.replace(sc_params, use_tc_tiling_on_sc=False)
    mesh = plsc.VectorSubcoreMesh(core_axis_name="c", subcore_axis_name="s")
    refine = pl.kernel(
        functools.partial(_sc_refine, n_chunk, d_feat, chunks_per_tile),
        out_type=jax.ShapeDtypeStruct((b,), jnp.float32),
        mesh=mesh,
        scratch_types=[
            pltpu.VMEM((n_chunk, d_feat), jnp.float32),
            pltpu.VMEM((n_chunk, d_feat), jnp.float32),
            pltpu.VMEM((n_proto, od), jnp.float32),
            pltpu.VMEM((d_feat, n_chunk), jnp.float32),
            pltpu.VMEM((n_chunk,), jnp.int32),
            pltpu.VMEM((n_chunk,), jnp.int32),
            pltpu.VMEM((d_feat,), jnp.float32),
            pltpu.VMEM((n_chunk,), jnp.float32),
            pltpu.SemaphoreType.DMA,
            pltpu.SemaphoreType.DMA,
            pltpu.SemaphoreType.DMA,
            pltpu.SemaphoreType.DMA,
            pltpu.SemaphoreType.DMA,
        ],
        compiler_params=sc_params,
    )
    out_flat = refine(prototypes, prototype_outputs, xr, win, relevance)
    return out_flat.reshape(b, od)
